# baseline (speedup 1.0000x reference)
import functools

import jax
import jax.numpy as jnp
from jax import lax
from jax.experimental import pallas as pl
from jax.experimental.pallas import tpu as pltpu
from jax.experimental.pallas import tpu_sc as plsc

N_NODES = 10000
N_EDGES = 320000
D_FEAT = 128
D_HID = 128
N_CLS = 16

L = 16
NC = 2
NS = 16
NW = NC * NS

E_SUPER = 1024
E_PER_W = 10240
E_PAD = E_PER_W * NW
ROWS_PER_TILE = N_NODES // NS


def _spmm_sc(h, src2d, dst2d, w, d, col_split):
    mesh = plsc.VectorSubcoreMesh(
        core_axis_name="c", subcore_axis_name="s", num_cores=NC, num_subcores=NS)
    nj = d // L
    e_blk = 256
    nk = e_blk // 128
    nq = E_SUPER // e_blk
    supers = (E_PAD // NS if col_split else E_PER_W) // E_SUPER

    @functools.partial(
        pl.kernel,
        out_type=jax.ShapeDtypeStruct((NC, NS, ROWS_PER_TILE, d), jnp.float32),
        mesh=mesh,
        scratch_types=[
            pltpu.VMEM((8, 128), jnp.int32),
            pltpu.VMEM((8, 128), jnp.int32),
            pltpu.VMEM((E_SUPER,), jnp.float32),
            pltpu.VMEM((e_blk, d), jnp.float32),
            pltpu.VMEM_SHARED((N_NODES, d), jnp.float32),
            pltpu.VMEM_SHARED((N_NODES, d), jnp.float32),
            pltpu.SemaphoreType.DMA,
        ],
        compiler_params=pltpu.CompilerParams(use_tc_tiling_on_sc=False),
    )
    def spmm_kernel(h_hbm, src_hbm, dst_hbm, w_hbm, out_hbm,
                    src_v, dst_v, w_v, rows_v, tab_sh, acc_sh, sem):
        c = lax.axis_index("c")
        s = lax.axis_index("s")

        def zrow(r, carry):
            for j in range(nj):
                rows_v[r, pl.ds(j * L, L)] = jnp.zeros((L,), jnp.float32)
            return carry
        lax.fori_loop(0, e_blk, zrow, 0)
        nbase = s * ROWS_PER_TILE
        done = 0
        while done < ROWS_PER_TILE:
            n = min(e_blk, ROWS_PER_TILE - done)
            pltpu.sync_copy(rows_v.at[pl.ds(0, n)], acc_sh.at[pl.ds(nbase + done, n)])
            done += n
        if col_split:
            pltpu.sync_copy(h_hbm.at[c, pl.ds(nbase, ROWS_PER_TILE)],
                            tab_sh.at[pl.ds(nbase, ROWS_PER_TILE)])
        else:
            pltpu.sync_copy(h_hbm.at[pl.ds(nbase, ROWS_PER_TILE)],
                            tab_sh.at[pl.ds(nbase, ROWS_PER_TILE)])
        plsc.subcore_barrier()

        if col_split:
            ebase = s * (E_PAD // NS)
        else:
            ebase = (c * NS + s) * E_PER_W
        rbase = ebase // 128

        def eblock(i, carry):
            roff = pl.multiple_of(rbase + i * 8, 8)
            pltpu.sync_copy(src_hbm.at[pl.ds(roff, 8)], src_v)
            pltpu.sync_copy(dst_hbm.at[pl.ds(roff, 8)], dst_v)
            pltpu.sync_copy(w_hbm.at[pl.ds(ebase + i * E_SUPER, E_SUPER)], w_v)
            for q in range(nq):
                cps = [
                    pltpu.async_copy(tab_sh.at[src_v.at[nk * q + k]],
                                     rows_v.at[pl.ds(k * 128, 128)], sem)
                    for k in range(nk)
                ]
                for cp in cps:
                    cp.wait()
                def scale(g, carry2):
                    wv16 = w_v[pl.ds(q * e_blk + g * L, L)]
                    for lane in range(L):
                        wb = jnp.full((L,), wv16[lane], dtype=jnp.float32)
                        e = g * L + lane
                        for j in range(nj):
                            sl = pl.ds(j * L, L)
                            rows_v[e, sl] = rows_v[e, sl] * wb
                    return carry2
                lax.fori_loop(0, e_blk // L, scale, 0)
                for k in range(nk):
                    pltpu.sync_copy(rows_v.at[pl.ds(k * 128, 128)],
                                    acc_sh.at[dst_v.at[nk * q + k]], add=True)
            return carry
        lax.fori_loop(0, supers, eblock, 0)
        plsc.subcore_barrier()

        pltpu.sync_copy(acc_sh.at[pl.ds(nbase, ROWS_PER_TILE)],
                        out_hbm.at[c, s])

    out = spmm_kernel(h, src2d, dst2d, w)
    return out.reshape(NC, N_NODES, d)


def _mm1_tc(x, W1, b1):
    def body(x_ref, wa_ref, wb_ref, ba_ref, bb_ref, o_ref):
        xv = x_ref[...]
        o_ref[0] = jnp.dot(xv, wa_ref[...],
                           preferred_element_type=jnp.float32) + ba_ref[...]
        o_ref[1] = jnp.dot(xv, wb_ref[...],
                           preferred_element_type=jnp.float32) + bb_ref[...]
    hd = D_HID // 2
    return pl.pallas_call(
        body,
        grid=(10,),
        in_specs=[
            pl.BlockSpec((1000, D_FEAT), lambda i: (i, 0)),
            pl.BlockSpec((D_FEAT, hd), lambda i: (0, 0)),
            pl.BlockSpec((D_FEAT, hd), lambda i: (0, 0)),
            pl.BlockSpec((1, hd), lambda i: (0, 0)),
            pl.BlockSpec((1, hd), lambda i: (0, 0)),
        ],
        out_specs=pl.BlockSpec((2, 1000, hd), lambda i: (0, i, 0)),
        out_shape=jax.ShapeDtypeStruct((NC, N_NODES, hd), jnp.float32),
    )(x, W1[:, :hd], W1[:, hd:], b1[:hd].reshape(1, hd), b1[hd:].reshape(1, hd))


def _mm2_tc(p, W2a, W2b, b2):
    def body(p_ref, wa_ref, wb_ref, b_ref, o_ref):
        ha = jnp.maximum(p_ref[0], 0.0)
        hb = jnp.maximum(p_ref[1], 0.0)
        o_ref[...] = (jnp.dot(ha, wa_ref[...], preferred_element_type=jnp.float32)
                      + jnp.dot(hb, wb_ref[...], preferred_element_type=jnp.float32)
                      + b_ref[...])
    return pl.pallas_call(
        body,
        grid=(10,),
        in_specs=[
            pl.BlockSpec((2, 1000, D_HID // 2), lambda i: (0, i, 0)),
            pl.BlockSpec((D_HID // 2, N_CLS), lambda i: (0, 0)),
            pl.BlockSpec((D_HID // 2, N_CLS), lambda i: (0, 0)),
            pl.BlockSpec((1, N_CLS), lambda i: (0, 0)),
        ],
        out_specs=pl.BlockSpec((1000, N_CLS), lambda i: (i, 0)),
        out_shape=jax.ShapeDtypeStruct((N_NODES, N_CLS), jnp.float32),
    )(p, W2a, W2b, b2.reshape(1, N_CLS))


def _add_tc(q):
    def body(q_ref, o_ref):
        o_ref[...] = q_ref[0] + q_ref[1]
    return pl.pallas_call(
        body,
        grid=(1,),
        in_specs=[pl.BlockSpec((2, N_NODES, N_CLS), lambda i: (0, 0, 0))],
        out_specs=pl.BlockSpec((N_NODES, N_CLS), lambda i: (0, 0)),
        out_shape=jax.ShapeDtypeStruct((N_NODES, N_CLS), jnp.float32),
    )(q)


def kernel(x, edge_index, edge_weight, W1, b1, W2, b2):
    dst = edge_index[0].astype(jnp.int32)
    src = edge_index[1].astype(jnp.int32)
    pad = E_PAD - N_EDGES
    zi = jnp.zeros((pad,), jnp.int32)
    src2d = jnp.concatenate([src, zi]).reshape(E_PAD // 128, 128)
    dst2d = jnp.concatenate([dst, zi]).reshape(E_PAD // 128, 128)
    wpad = jnp.concatenate([edge_weight, jnp.zeros((pad,), jnp.float32)])

    h1 = _mm1_tc(x, W1, b1)
    p = _spmm_sc(h1, src2d, dst2d, wpad, D_HID // 2, col_split=True)
    h2 = _mm2_tc(p, W2[:D_HID // 2], W2[D_HID // 2:], b2)
    q = _spmm_sc(h2, src2d, dst2d, wpad, N_CLS, col_split=False)
    return _add_tc(q)


# speedup vs baseline: 1.6717x; 1.6717x over previous
import functools

import jax
import jax.numpy as jnp
from jax import lax
from jax.experimental import pallas as pl
from jax.experimental.pallas import tpu as pltpu
from jax.experimental.pallas import tpu_sc as plsc

N_NODES = 10000
N_EDGES = 320000
D_FEAT = 128
D_HID = 128
N_CLS = 16

L = 16
NC = 2
NS = 16
NW = NC * NS

E_SUPER = 1024
E_PER_W = 10240
E_PAD = E_PER_W * NW
ROWS_PER_TILE = N_NODES // NS


def _spmm_sc(h, src2d, dst2d, w, d, col_split):
    mesh = plsc.VectorSubcoreMesh(
        core_axis_name="c", subcore_axis_name="s", num_cores=NC, num_subcores=NS)
    nj = d // L
    e_blk = 128
    nq = E_SUPER // e_blk
    supers = (E_PAD // NS if col_split else E_PER_W) // E_SUPER
    assert supers % 2 == 0

    @functools.partial(
        pl.kernel,
        out_type=jax.ShapeDtypeStruct((NC, NS, ROWS_PER_TILE, d), jnp.float32),
        mesh=mesh,
        scratch_types=[
            pltpu.VMEM((2, 8, 128), jnp.int32),
            pltpu.VMEM((2, 8, 128), jnp.int32),
            pltpu.VMEM((2, E_SUPER), jnp.float32),
            pltpu.VMEM((4, e_blk, d), jnp.float32),
            pltpu.VMEM_SHARED((N_NODES, d), jnp.float32),
            pltpu.VMEM_SHARED((N_NODES, d), jnp.float32),
            pltpu.SemaphoreType.DMA,
            pltpu.SemaphoreType.DMA,
        ],
        compiler_params=pltpu.CompilerParams(use_tc_tiling_on_sc=False),
    )
    def spmm_kernel(h_hbm, src_hbm, dst_hbm, w_hbm, out_hbm,
                    src_v, dst_v, w_v, rows_v, tab_sh, acc_sh,
                    sem_g, sem_s):
        c = lax.axis_index("c")
        s = lax.axis_index("s")

        def zslot(slot):
            def zr(r, carry):
                for j in range(nj):
                    rows_v[slot, r, pl.ds(j * L, L)] = jnp.zeros((L,), jnp.float32)
                return carry
            lax.fori_loop(0, e_blk, zr, 0)
        for slot in range(4):
            zslot(slot)
        nbase = s * ROWS_PER_TILE
        done = 0
        while done < ROWS_PER_TILE:
            n = min(e_blk, ROWS_PER_TILE - done)
            pltpu.sync_copy(rows_v.at[0, pl.ds(0, n)],
                            acc_sh.at[pl.ds(nbase + done, n)])
            done += n
        if col_split:
            pltpu.sync_copy(h_hbm.at[c, pl.ds(nbase, ROWS_PER_TILE)],
                            tab_sh.at[pl.ds(nbase, ROWS_PER_TILE)])
        else:
            pltpu.sync_copy(h_hbm.at[pl.ds(nbase, ROWS_PER_TILE)],
                            tab_sh.at[pl.ds(nbase, ROWS_PER_TILE)])
        plsc.subcore_barrier()

        if col_split:
            ebase = s * (E_PAD // NS)
        else:
            ebase = (c * NS + s) * E_PER_W
        rbase = ebase // 128

        def load_super(sp, hb):
            sp = lax.rem(sp, supers)
            roff = pl.multiple_of(rbase + sp * 8, 8)
            pltpu.sync_copy(src_hbm.at[pl.ds(roff, 8)], src_v.at[hb])
            pltpu.sync_copy(dst_hbm.at[pl.ds(roff, 8)], dst_v.at[hb])
            pltpu.sync_copy(w_hbm.at[pl.ds(ebase + sp * E_SUPER, E_SUPER)],
                            w_v.at[hb])

        def fire_gather(hb, q, slot):
            pltpu.async_copy(tab_sh.at[src_v.at[hb, q]],
                             rows_v.at[slot], sem_g)

        def fire_scatter(hb, q, slot):
            pltpu.async_copy(rows_v.at[slot],
                             acc_sh.at[dst_v.at[hb, q]], sem_s, add=True)

        def wait_gather():
            pltpu.make_async_copy(tab_sh.at[src_v.at[0, 0]],
                                  rows_v.at[0], sem_g).wait()

        def wait_scatter():
            pltpu.make_async_copy(rows_v.at[0],
                                  acc_sh.at[dst_v.at[0, 0]], sem_s).wait()

        def scale(slot, hb, q):
            def sgrp(g, carry):
                wv16 = w_v[hb, pl.ds(q * e_blk + g * L, L)]
                for lane in range(L):
                    wb = jnp.full((L,), wv16[lane], dtype=jnp.float32)
                    e = g * L + lane
                    for j in range(nj):
                        sl = pl.ds(j * L, L)
                        rows_v[slot, e, sl] = rows_v[slot, e, sl] * wb
                return carry
            lax.fori_loop(0, e_blk // L, sgrp, 0)

        load_super(0, 0)
        fire_gather(0, 0, 0)
        fire_gather(0, 1, 1)
        fire_scatter(0, 0, 2)
        fire_scatter(0, 1, 3)

        def pair(u, carry):
            for hb in range(2):
                for q in range(nq):
                    slot = q % 4
                    wait_gather()
                    scale(slot, hb, q)
                    fire_scatter(hb, q, slot)
                    wait_scatter()
                    if q == 2:
                        load_super(2 * u + hb + 1, 1 - hb)
                    nslot = (q + 2) % 4
                    if q < nq - 2:
                        fire_gather(hb, q + 2, nslot)
                    else:
                        fire_gather(1 - hb, q - (nq - 2), nslot)
            return carry
        lax.fori_loop(0, supers // 2, pair, 0)
        wait_gather()
        wait_gather()
        wait_scatter()
        wait_scatter()
        plsc.subcore_barrier()

        pltpu.sync_copy(acc_sh.at[pl.ds(nbase, ROWS_PER_TILE)],
                        out_hbm.at[c, s])

    out = spmm_kernel(h, src2d, dst2d, w)
    return out.reshape(NC, N_NODES, d)


def _mm1_tc(x, W1, b1):
    def body(x_ref, wa_ref, wb_ref, ba_ref, bb_ref, o_ref):
        xv = x_ref[...]
        o_ref[0] = jnp.dot(xv, wa_ref[...],
                           preferred_element_type=jnp.float32) + ba_ref[...]
        o_ref[1] = jnp.dot(xv, wb_ref[...],
                           preferred_element_type=jnp.float32) + bb_ref[...]
    hd = D_HID // 2
    return pl.pallas_call(
        body,
        grid=(10,),
        in_specs=[
            pl.BlockSpec((1000, D_FEAT), lambda i: (i, 0)),
            pl.BlockSpec((D_FEAT, hd), lambda i: (0, 0)),
            pl.BlockSpec((D_FEAT, hd), lambda i: (0, 0)),
            pl.BlockSpec((1, hd), lambda i: (0, 0)),
            pl.BlockSpec((1, hd), lambda i: (0, 0)),
        ],
        out_specs=pl.BlockSpec((2, 1000, hd), lambda i: (0, i, 0)),
        out_shape=jax.ShapeDtypeStruct((NC, N_NODES, hd), jnp.float32),
    )(x, W1[:, :hd], W1[:, hd:], b1[:hd].reshape(1, hd), b1[hd:].reshape(1, hd))


def _mm2_tc(p, W2a, W2b, b2):
    def body(p_ref, wa_ref, wb_ref, b_ref, o_ref):
        ha = jnp.maximum(p_ref[0], 0.0)
        hb = jnp.maximum(p_ref[1], 0.0)
        o_ref[...] = (jnp.dot(ha, wa_ref[...], preferred_element_type=jnp.float32)
                      + jnp.dot(hb, wb_ref[...], preferred_element_type=jnp.float32)
                      + b_ref[...])
    return pl.pallas_call(
        body,
        grid=(10,),
        in_specs=[
            pl.BlockSpec((2, 1000, D_HID // 2), lambda i: (0, i, 0)),
            pl.BlockSpec((D_HID // 2, N_CLS), lambda i: (0, 0)),
            pl.BlockSpec((D_HID // 2, N_CLS), lambda i: (0, 0)),
            pl.BlockSpec((1, N_CLS), lambda i: (0, 0)),
        ],
        out_specs=pl.BlockSpec((1000, N_CLS), lambda i: (i, 0)),
        out_shape=jax.ShapeDtypeStruct((N_NODES, N_CLS), jnp.float32),
    )(p, W2a, W2b, b2.reshape(1, N_CLS))


def _add_tc(q):
    def body(q_ref, o_ref):
        o_ref[...] = q_ref[0] + q_ref[1]
    return pl.pallas_call(
        body,
        grid=(1,),
        in_specs=[pl.BlockSpec((2, N_NODES, N_CLS), lambda i: (0, 0, 0))],
        out_specs=pl.BlockSpec((N_NODES, N_CLS), lambda i: (0, 0)),
        out_shape=jax.ShapeDtypeStruct((N_NODES, N_CLS), jnp.float32),
    )(q)


def kernel(x, edge_index, edge_weight, W1, b1, W2, b2):
    dst = edge_index[0].astype(jnp.int32)
    src = edge_index[1].astype(jnp.int32)
    pad = E_PAD - N_EDGES
    zi = jnp.zeros((pad,), jnp.int32)
    src2d = jnp.concatenate([src, zi]).reshape(E_PAD // 128, 128)
    dst2d = jnp.concatenate([dst, zi]).reshape(E_PAD // 128, 128)
    wpad = jnp.concatenate([edge_weight, jnp.zeros((pad,), jnp.float32)])

    h1 = _mm1_tc(x, W1, b1)
    p = _spmm_sc(h1, src2d, dst2d, wpad, D_HID // 2, col_split=True)
    h2 = _mm2_tc(p, W2[:D_HID // 2], W2[D_HID // 2:], b2)
    q = _spmm_sc(h2, src2d, dst2d, wpad, N_CLS, col_split=False)
    return _add_tc(q)


# speedup vs baseline: 2.0686x; 1.2374x over previous
import functools

import jax
import jax.numpy as jnp
from jax import lax
from jax.experimental import pallas as pl
from jax.experimental.pallas import tpu as pltpu
from jax.experimental.pallas import tpu_sc as plsc

N_NODES = 10000
N_EDGES = 320000
D_FEAT = 128
D_HID = 128
N_CLS = 16

L = 16
NC = 2
NS = 16
NW = NC * NS

E_SUPER = 1024
E_PER_W = 10240
E_PAD = E_PER_W * NW
ROWS_PER_TILE = N_NODES // NS


def _spmm_sc(h, src2d, dst2d, w, d, col_split):
    mesh = plsc.VectorSubcoreMesh(
        core_axis_name="c", subcore_axis_name="s", num_cores=NC, num_subcores=NS)
    nj = d // L
    e_blk = 128
    nq = E_SUPER // e_blk
    supers = (E_PAD // NS if col_split else E_PER_W) // E_SUPER
    assert supers % 2 == 0

    @functools.partial(
        pl.kernel,
        out_type=jax.ShapeDtypeStruct((NC, NS, ROWS_PER_TILE, d), jnp.float32),
        mesh=mesh,
        scratch_types=[
            pltpu.VMEM((2, 8, 128), jnp.int32),
            pltpu.VMEM((2, 8, 128), jnp.int32),
            pltpu.VMEM((2, E_SUPER), jnp.float32),
            pltpu.VMEM((4, e_blk, d), jnp.float32),
            pltpu.VMEM_SHARED((N_NODES, d), jnp.float32),
            pltpu.VMEM_SHARED((N_NODES, d), jnp.float32),
            pltpu.SemaphoreType.DMA,
            pltpu.SemaphoreType.DMA,
        ],
        compiler_params=pltpu.CompilerParams(use_tc_tiling_on_sc=False),
    )
    def spmm_kernel(h_hbm, src_hbm, dst_hbm, w_hbm, out_hbm,
                    src_v, dst_v, w_v, rows_v, tab_sh, acc_sh,
                    sem_g, sem_s):
        c = lax.axis_index("c")
        s = lax.axis_index("s")

        def zslot(slot):
            def zr(r, carry):
                for j in range(nj):
                    rows_v[slot, r, pl.ds(j * L, L)] = jnp.zeros((L,), jnp.float32)
                return carry
            lax.fori_loop(0, e_blk, zr, 0)
        for slot in range(4):
            zslot(slot)
        nbase = s * ROWS_PER_TILE
        done = 0
        while done < ROWS_PER_TILE:
            n = min(e_blk, ROWS_PER_TILE - done)
            pltpu.sync_copy(rows_v.at[0, pl.ds(0, n)],
                            acc_sh.at[pl.ds(nbase + done, n)])
            done += n
        if col_split:
            pltpu.sync_copy(h_hbm.at[c, pl.ds(nbase, ROWS_PER_TILE)],
                            tab_sh.at[pl.ds(nbase, ROWS_PER_TILE)])
        else:
            pltpu.sync_copy(h_hbm.at[pl.ds(nbase, ROWS_PER_TILE)],
                            tab_sh.at[pl.ds(nbase, ROWS_PER_TILE)])
        plsc.subcore_barrier()

        if col_split:
            ebase = s * (E_PAD // NS)
        else:
            ebase = (c * NS + s) * E_PER_W
        rbase = ebase // 128

        def load_super(sp, hb):
            sp = lax.rem(sp, supers)
            roff = pl.multiple_of(rbase + sp * 8, 8)
            pltpu.sync_copy(src_hbm.at[pl.ds(roff, 8)], src_v.at[hb])
            pltpu.sync_copy(dst_hbm.at[pl.ds(roff, 8)], dst_v.at[hb])
            pltpu.sync_copy(w_hbm.at[pl.ds(ebase + sp * E_SUPER, E_SUPER)],
                            w_v.at[hb])

        def fire_gather(hb, q, slot):
            pltpu.async_copy(tab_sh.at[src_v.at[hb, q]],
                             rows_v.at[slot], sem_g)

        def fire_scatter(hb, q, slot):
            pltpu.async_copy(rows_v.at[slot],
                             acc_sh.at[dst_v.at[hb, q]], sem_s, add=True)

        def wait_gather():
            pltpu.make_async_copy(tab_sh.at[src_v.at[0, 0]],
                                  rows_v.at[0], sem_g).wait()

        def wait_scatter():
            pltpu.make_async_copy(rows_v.at[0],
                                  acc_sh.at[dst_v.at[0, 0]], sem_s).wait()

        def scale(slot, hb, q):
            @plsc.parallel_loop(0, e_blk // L, 1, unroll=2)
            def sgrp(g):
                wv16 = w_v[hb, pl.ds(q * e_blk + g * L, L)]
                for lane in range(L):
                    wb = jnp.full((L,), wv16[lane], dtype=jnp.float32)
                    e = g * L + lane
                    for j in range(nj):
                        sl = pl.ds(j * L, L)
                        rows_v[slot, e, sl] = rows_v[slot, e, sl] * wb

        load_super(0, 0)
        fire_gather(0, 0, 0)
        fire_gather(0, 1, 1)
        fire_scatter(0, 0, 2)
        fire_scatter(0, 1, 3)

        def pair(u, carry):
            for hb in range(2):
                for q in range(nq):
                    slot = q % 4
                    wait_gather()
                    wait_scatter()
                    nslot = (q + 2) % 4
                    if q < nq - 2:
                        fire_gather(hb, q + 2, nslot)
                    else:
                        fire_gather(1 - hb, q - (nq - 2), nslot)
                    if q == 2:
                        load_super(2 * u + hb + 1, 1 - hb)
                    scale(slot, hb, q)
                    fire_scatter(hb, q, slot)
            return carry
        lax.fori_loop(0, supers // 2, pair, 0)
        wait_gather()
        wait_gather()
        wait_scatter()
        wait_scatter()
        plsc.subcore_barrier()

        pltpu.sync_copy(acc_sh.at[pl.ds(nbase, ROWS_PER_TILE)],
                        out_hbm.at[c, s])

    out = spmm_kernel(h, src2d, dst2d, w)
    return out.reshape(NC, N_NODES, d)


def _mm1_tc(x, W1, b1):
    def body(x_ref, wa_ref, wb_ref, ba_ref, bb_ref, o_ref):
        xv = x_ref[...]
        o_ref[0] = jnp.dot(xv, wa_ref[...],
                           preferred_element_type=jnp.float32) + ba_ref[...]
        o_ref[1] = jnp.dot(xv, wb_ref[...],
                           preferred_element_type=jnp.float32) + bb_ref[...]
    hd = D_HID // 2
    return pl.pallas_call(
        body,
        grid=(10,),
        in_specs=[
            pl.BlockSpec((1000, D_FEAT), lambda i: (i, 0)),
            pl.BlockSpec((D_FEAT, hd), lambda i: (0, 0)),
            pl.BlockSpec((D_FEAT, hd), lambda i: (0, 0)),
            pl.BlockSpec((1, hd), lambda i: (0, 0)),
            pl.BlockSpec((1, hd), lambda i: (0, 0)),
        ],
        out_specs=pl.BlockSpec((2, 1000, hd), lambda i: (0, i, 0)),
        out_shape=jax.ShapeDtypeStruct((NC, N_NODES, hd), jnp.float32),
    )(x, W1[:, :hd], W1[:, hd:], b1[:hd].reshape(1, hd), b1[hd:].reshape(1, hd))


def _mm2_tc(p, W2a, W2b, b2):
    def body(p_ref, wa_ref, wb_ref, b_ref, o_ref):
        ha = jnp.maximum(p_ref[0], 0.0)
        hb = jnp.maximum(p_ref[1], 0.0)
        o_ref[...] = (jnp.dot(ha, wa_ref[...], preferred_element_type=jnp.float32)
                      + jnp.dot(hb, wb_ref[...], preferred_element_type=jnp.float32)
                      + b_ref[...])
    return pl.pallas_call(
        body,
        grid=(10,),
        in_specs=[
            pl.BlockSpec((2, 1000, D_HID // 2), lambda i: (0, i, 0)),
            pl.BlockSpec((D_HID // 2, N_CLS), lambda i: (0, 0)),
            pl.BlockSpec((D_HID // 2, N_CLS), lambda i: (0, 0)),
            pl.BlockSpec((1, N_CLS), lambda i: (0, 0)),
        ],
        out_specs=pl.BlockSpec((1000, N_CLS), lambda i: (i, 0)),
        out_shape=jax.ShapeDtypeStruct((N_NODES, N_CLS), jnp.float32),
    )(p, W2a, W2b, b2.reshape(1, N_CLS))


def _add_tc(q):
    def body(q_ref, o_ref):
        o_ref[...] = q_ref[0] + q_ref[1]
    return pl.pallas_call(
        body,
        grid=(1,),
        in_specs=[pl.BlockSpec((2, N_NODES, N_CLS), lambda i: (0, 0, 0))],
        out_specs=pl.BlockSpec((N_NODES, N_CLS), lambda i: (0, 0)),
        out_shape=jax.ShapeDtypeStruct((N_NODES, N_CLS), jnp.float32),
    )(q)


def kernel(x, edge_index, edge_weight, W1, b1, W2, b2):
    dst = edge_index[0].astype(jnp.int32)
    src = edge_index[1].astype(jnp.int32)
    pad = E_PAD - N_EDGES
    zi = jnp.zeros((pad,), jnp.int32)
    src2d = jnp.concatenate([src, zi]).reshape(E_PAD // 128, 128)
    dst2d = jnp.concatenate([dst, zi]).reshape(E_PAD // 128, 128)
    wpad = jnp.concatenate([edge_weight, jnp.zeros((pad,), jnp.float32)])

    h1 = _mm1_tc(x, W1, b1)
    p = _spmm_sc(h1, src2d, dst2d, wpad, D_HID // 2, col_split=True)
    h2 = _mm2_tc(p, W2[:D_HID // 2], W2[D_HID // 2:], b2)
    q = _spmm_sc(h2, src2d, dst2d, wpad, N_CLS, col_split=False)
    return _add_tc(q)
